# Initial kernel scaffold; baseline (speedup 1.0000x reference)
import jax
import jax.numpy as jnp
from jax.experimental import pallas as pl


def kernel(Gu, Gi, edge_index):
    raise NotImplementedError("write your pallas kernel here")


# speedup vs baseline: 11.1623x; 11.1623x over previous
import functools

import jax
import jax.numpy as jnp
from jax import lax
from jax.experimental import pallas as pl
from jax.experimental.pallas import tpu as pltpu
from jax.experimental.pallas import tpu_sc as plsc

N_USERS = 10000
N_NODES = 50000
N_PAD = 50176
DUMP = N_PAD - N_NODES
K = 64
KQ = 16
NQ = 4
E = 800000
W = 1024
NW = 800
E_PAD = NW * W
WPT = NW // 16
WPT_DEG = NW // 32
TILE_ROWS = N_PAD // 16

_mesh = plsc.VectorSubcoreMesh(core_axis_name="c", subcore_axis_name="s")
_sc_params = pltpu.CompilerParams(use_tc_tiling_on_sc=False)


@functools.partial(
    pl.kernel,
    out_type=jax.ShapeDtypeStruct((2 * N_PAD,), jnp.float32),
    mesh=_mesh,
    compiler_params=_sc_params,
    scratch_types=[
        pltpu.VMEM((8, 128), jnp.int32),
        pltpu.VMEM((8, 128), jnp.float32),
        pltpu.VMEM((TILE_ROWS,), jnp.float32),
        pltpu.VMEM_SHARED((N_PAD,), jnp.float32),
        pltpu.SemaphoreType.DMA,
    ],
)
def _deg_kernel(col_hbm, zrow_hbm, deg_out, col_v, ones_v, stage_v, deg_sh,
                sem):
    c = lax.axis_index("c")
    s = lax.axis_index("s")
    for j in range(8):
        for k in range(8):
            ones_v[j, pl.ds(k * 16, 16)] = jnp.full((16,), 1.0, jnp.float32)
    pltpu.sync_copy(zrow_hbm, stage_v)
    pltpu.sync_copy(stage_v, deg_sh.at[pl.ds(s * TILE_ROWS, TILE_ROWS)])
    plsc.subcore_barrier()

    def body(w, carry):
        g = c * (16 * WPT_DEG) + s * WPT_DEG + w
        pltpu.sync_copy(col_hbm.at[g], col_v)
        hs = [
            pltpu.async_copy(ones_v.at[j], deg_sh.at[col_v.at[j]], sem,
                             add=True)
            for j in range(8)
        ]
        for h in hs:
            h.wait()
        return carry

    lax.fori_loop(0, WPT_DEG, body, 0)
    plsc.subcore_barrier()
    pltpu.sync_copy(deg_sh.at[pl.ds(s * TILE_ROWS, TILE_ROWS)], stage_v)
    pltpu.sync_copy(
        stage_v, deg_out.at[pl.ds(c * N_PAD + s * TILE_ROWS, TILE_ROWS)]
    )


@functools.partial(
    pl.kernel,
    out_type=jax.ShapeDtypeStruct((NQ * N_PAD, KQ), jnp.float32),
    mesh=_mesh,
    compiler_params=_sc_params,
    scratch_types=[
        pltpu.VMEM((8, 128), jnp.int32),
        pltpu.VMEM((8, 128), jnp.int32),
        pltpu.VMEM((W, KQ), jnp.float32),
        pltpu.VMEM_SHARED((N_PAD, KQ), jnp.float32),
        pltpu.SemaphoreType.DMA,
        pltpu.SemaphoreType.DMA,
    ],
)
def _layer_kernel(xs_hbm, row4_hbm, col_hbm, ztile_hbm, acc_out,
                  row_v, col_v, msg_v, acc_sh, gsem, ssem):
    c = lax.axis_index("c")
    s = lax.axis_index("s")
    base = s * TILE_ROWS

    for p in range(2):
        q = 2 * c + p
        pltpu.sync_copy(ztile_hbm, msg_v)
        for k in range(3):
            pltpu.sync_copy(msg_v, acc_sh.at[pl.ds(base + k * W, W), :])
        pltpu.sync_copy(
            msg_v.at[pl.ds(0, 64), :], acc_sh.at[pl.ds(base + 3 * W, 64), :]
        )
        plsc.subcore_barrier()

        def body(w, carry):
            g = s * WPT + w
            pltpu.sync_copy(row4_hbm.at[q * NW + g], row_v)
            pltpu.sync_copy(col_hbm.at[g], col_v)
            gh = [
                pltpu.async_copy(
                    xs_hbm.at[row_v.at[j]],
                    msg_v.at[pl.ds(j * 128, 128), :], gsem,
                )
                for j in range(8)
            ]
            for h in gh:
                h.wait()
            sh = [
                pltpu.async_copy(
                    msg_v.at[pl.ds(j * 128, 128), :],
                    acc_sh.at[col_v.at[j]], ssem, add=True,
                )
                for j in range(8)
            ]
            for h in sh:
                h.wait()
            return carry

        lax.fori_loop(0, WPT, body, 0)
        plsc.subcore_barrier()
        obase = q * N_PAD + base
        for k in range(3):
            pltpu.sync_copy(acc_sh.at[pl.ds(base + k * W, W), :], msg_v)
            pltpu.sync_copy(msg_v, acc_out.at[pl.ds(obase + k * W, W), :])
        pltpu.sync_copy(
            acc_sh.at[pl.ds(base + 3 * W, 64), :], msg_v.at[pl.ds(0, 64), :]
        )
        pltpu.sync_copy(
            msg_v.at[pl.ds(0, 64), :],
            acc_out.at[pl.ds(obase + 3 * W, 64), :],
        )


_B = TILE_ROWS


def _prep_body(deg_ref, x_ref, xs_ref, d2_ref, dinvb_ref):
    d = deg_ref[0] + deg_ref[1]
    dinv = jnp.where(d > 0, lax.rsqrt(jnp.maximum(d, 1e-12)), 0.0)
    dinvb = jnp.broadcast_to(dinv, (_B, K))
    xs = x_ref[...] * dinvb
    d2b = jnp.broadcast_to(dinv * dinv, (_B, KQ))
    for q in range(NQ):
        xs_ref[q] = xs[:, q * KQ:(q + 1) * KQ]
        d2_ref[q] = d2b
    dinvb_ref[...] = dinvb


_prep = pl.pallas_call(
    _prep_body,
    grid=(N_PAD // _B,),
    in_specs=[
        pl.BlockSpec((2, _B, 1), lambda i: (0, i, 0)),
        pl.BlockSpec((_B, K), lambda i: (i, 0)),
    ],
    out_specs=[
        pl.BlockSpec((NQ, _B, KQ), lambda i: (0, i, 0)),
        pl.BlockSpec((NQ, _B, KQ), lambda i: (0, i, 0)),
        pl.BlockSpec((_B, K), lambda i: (i, 0)),
    ],
    out_shape=[
        jax.ShapeDtypeStruct((NQ, N_PAD, KQ), jnp.float32),
        jax.ShapeDtypeStruct((NQ, N_PAD, KQ), jnp.float32),
        jax.ShapeDtypeStruct((N_PAD, K), jnp.float32),
    ],
)


def _scale_body(a_ref, d2_ref, o_ref):
    o_ref[...] = a_ref[...] * d2_ref[...]


_scale = pl.pallas_call(
    _scale_body,
    grid=(NQ * N_PAD // _B,),
    in_specs=[
        pl.BlockSpec((_B, KQ), lambda i: (i, 0)),
        pl.BlockSpec((_B, KQ), lambda i: (i, 0)),
    ],
    out_specs=pl.BlockSpec((_B, KQ), lambda i: (i, 0)),
    out_shape=jax.ShapeDtypeStruct((NQ * N_PAD, KQ), jnp.float32),
)


def _final_body(a1_ref, a2_ref, a3_ref, dinvb_ref, o_ref):
    third = jnp.float32(1.0 / 3.0)
    for q in range(NQ):
        sq = a1_ref[q] + a2_ref[q] + a3_ref[q]
        o_ref[:, q * KQ:(q + 1) * KQ] = (
            sq * dinvb_ref[:, q * KQ:(q + 1) * KQ] * third
        )


_final = pl.pallas_call(
    _final_body,
    grid=(N_PAD // _B,),
    in_specs=[
        pl.BlockSpec((NQ, _B, KQ), lambda i: (0, i, 0)),
        pl.BlockSpec((NQ, _B, KQ), lambda i: (0, i, 0)),
        pl.BlockSpec((NQ, _B, KQ), lambda i: (0, i, 0)),
        pl.BlockSpec((_B, K), lambda i: (i, 0)),
    ],
    out_specs=pl.BlockSpec((_B, K), lambda i: (i, 0)),
    out_shape=jax.ShapeDtypeStruct((N_PAD, K), jnp.float32),
)


def kernel(Gu, Gi, edge_index):
    x0 = jnp.concatenate([Gu, Gi], axis=0)
    x0p = jnp.pad(x0, ((0, N_PAD - N_NODES), (0, 0)))

    row = edge_index[0]
    col = edge_index[1]
    npad = E_PAD - E
    i = jnp.arange(npad, dtype=jnp.int32)
    prow = i % N_NODES
    pcol = N_NODES + (i % DUMP)
    rowp = jnp.concatenate([row, prow])
    colp = jnp.concatenate([col, pcol]).reshape(NW, 8, 128)
    row4 = jnp.stack([rowp + q * N_PAD for q in range(NQ)])
    row4 = row4.reshape(NQ * NW, 8, 128)

    zrow = jnp.zeros((TILE_ROWS,), jnp.float32)
    ztile = jnp.zeros((W, KQ), jnp.float32)

    deg = _deg_kernel(colp, zrow)
    xs1, d2, dinvb = _prep(deg.reshape(2, N_PAD, 1), x0p)
    d2f = d2.reshape(NQ * N_PAD, KQ)

    acc1 = _layer_kernel(xs1.reshape(NQ * N_PAD, KQ), row4, colp, ztile)
    xs2 = _scale(acc1, d2f)
    acc2 = _layer_kernel(xs2, row4, colp, ztile)
    xs3 = _scale(acc2, d2f)
    acc3 = _layer_kernel(xs3, row4, colp, ztile)

    out = _final(
        acc1.reshape(NQ, N_PAD, KQ),
        acc2.reshape(NQ, N_PAD, KQ),
        acc3.reshape(NQ, N_PAD, KQ),
        dinvb,
    )
    return out[:N_USERS], out[N_USERS:N_NODES]


# speedup vs baseline: 19.2016x; 1.7202x over previous
import functools

import jax
import jax.numpy as jnp
from jax import lax
from jax.experimental import pallas as pl
from jax.experimental.pallas import tpu as pltpu
from jax.experimental.pallas import tpu_sc as plsc

N_USERS = 10000
N_NODES = 50000
N_PAD = 50176
DUMP = N_PAD - N_NODES
K = 64
KQ = 16
NQ = 4
E = 800000
W = 512
NS = W // 128
NW = 1600
E_PAD = NW * W
WPT = NW // 16
TILE_ROWS = N_PAD // 16
NFULL = TILE_ROWS // W
REM = TILE_ROWS - NFULL * W

_mesh = plsc.VectorSubcoreMesh(core_axis_name="c", subcore_axis_name="s")
_sc_params = pltpu.CompilerParams(use_tc_tiling_on_sc=False,
                                  needs_layout_passes=False)


@functools.partial(
    pl.kernel,
    out_type=[
        jax.ShapeDtypeStruct((N_PAD, K), jnp.float32),
        jax.ShapeDtypeStruct((NQ * N_PAD, KQ), jnp.float32),
        jax.ShapeDtypeStruct((NQ * N_PAD, KQ), jnp.float32),
        jax.ShapeDtypeStruct((NQ * N_PAD, KQ), jnp.float32),
    ],
    mesh=_mesh,
    compiler_params=_sc_params,
    scratch_types=[
        pltpu.VMEM((NS, 128), jnp.int32),
        pltpu.VMEM((NS, 128), jnp.int32),
        pltpu.VMEM((WPT, NS, 128), jnp.int32),
        pltpu.VMEM((W, KQ), jnp.float32),
        pltpu.VMEM((W, KQ), jnp.float32),
        pltpu.VMEM((NS, 128), jnp.float32),
        pltpu.VMEM((TILE_ROWS,), jnp.float32),
        pltpu.VMEM((TILE_ROWS,), jnp.float32),
        pltpu.VMEM_SHARED((N_PAD, KQ), jnp.float32),
        pltpu.VMEM_SHARED((N_PAD,), jnp.float32),
        pltpu.SemaphoreType.DMA,
        pltpu.SemaphoreType.DMA,
        pltpu.SemaphoreType.DMA,
    ],
)
def _main_kernel(x0_hbm, row4_hbm, col_hbm, ztile_hbm, zrow_hbm,
                 out_hbm, xs_scr, acc1_out, acc2_out,
                 row_v0, row_v1, col_all, msg_v0, msg_v1, ones_v,
                 stage_v, dinv_v, acc_sh, deg_sh,
                 gsem, ssem, isem):
    c = lax.axis_index("c")
    s = lax.axis_index("s")
    base = s * TILE_ROWS
    g0 = s * WPT
    third = jnp.float32(1.0 / 3.0)

    pltpu.sync_copy(col_hbm.at[pl.ds(g0, WPT)], col_all)

    for j in range(NS):
        for k in range(8):
            ones_v[j, pl.ds(k * 16, 16)] = jnp.full((16,), 1.0, jnp.float32)
    pltpu.sync_copy(zrow_hbm, stage_v)
    pltpu.sync_copy(stage_v, deg_sh.at[pl.ds(base, TILE_ROWS)])
    plsc.subcore_barrier()

    def deg_body(w, carry):
        hs = [
            pltpu.async_copy(ones_v.at[j], deg_sh.at[col_all.at[w, j]],
                             ssem, add=True)
            for j in range(NS)
        ]
        for h in hs:
            h.wait()
        return carry

    lax.fori_loop(0, WPT, deg_body, 0)
    plsc.subcore_barrier()

    pltpu.sync_copy(deg_sh.at[pl.ds(base, TILE_ROWS)], stage_v)

    def rsqrt_body(i, carry):
        x = stage_v[pl.ds(i * 16, 16)]
        h = x * jnp.float32(0.5)
        yi = jnp.int32(0x5F3759DF) - (plsc.bitcast(x, jnp.int32) >> 1)
        y = plsc.bitcast(yi, jnp.float32)
        for _ in range(3):
            y = y * (jnp.float32(1.5) - h * y * y)
        dinv_v[pl.ds(i * 16, 16)] = jnp.where(x > 0, y, jnp.float32(0.0))
        return carry

    lax.fori_loop(0, TILE_ROWS // 16, rsqrt_body, 0)

    def bcast(idx):
        return plsc.load_gather(dinv_v, [jnp.full((16,), 0, jnp.int32) + idx])

    def scale_rows(rows, mul_fn):
        def grp(g, carry):
            for u in range(16):
                r = g * 16 + u
                msg_v1[r, :] = mul_fn(r)
            return carry
        lax.fori_loop(0, rows // 16, grp, 0)

    def xs1_pass(p, carry):
        q = 2 * c + p

        def xs1_chunk(k, carry2, rows):
            lo = base + k * W
            pltpu.sync_copy(
                x0_hbm.at[pl.ds(lo, rows), pl.ds(q * KQ, KQ)],
                msg_v0.at[pl.ds(0, rows), :],
            )
            scale_rows(rows, lambda r: msg_v0[r, :] * bcast(k * W + r))
            pltpu.sync_copy(
                msg_v1.at[pl.ds(0, rows), :],
                xs_scr.at[pl.ds(q * N_PAD + lo, rows), :],
            )
            return carry2

        lax.fori_loop(0, NFULL, functools.partial(xs1_chunk, rows=W), 0)
        xs1_chunk(NFULL, 0, rows=REM)
        return carry

    lax.fori_loop(0, 2, xs1_pass, 0)

    def fire_gathers(rv, mv):
        for j in range(NS):
            pltpu.async_copy(
                xs_scr.at[rv.at[j]], mv.at[pl.ds(j * 128, 128), :], gsem
            )

    def fire_scatters(w, mv):
        for j in range(NS):
            pltpu.async_copy(
                mv.at[pl.ds(j * 128, 128), :],
                acc_sh.at[col_all.at[w, j]], ssem, add=True,
            )

    def drain(sem, dst):
        pltpu.make_async_copy(x0_hbm.at[pl.ds(0, W), pl.ds(0, KQ)], dst,
                              sem).wait()

    def scatter_pass(q):
        pltpu.sync_copy(row4_hbm.at[q * NW + g0], row_v0)
        fire_gathers(row_v0, msg_v0)

        def body2(i, carry):
            bufs = ((row_v0, msg_v0), (row_v1, msg_v1))
            for half in (0, 1):
                w = 2 * i + half
                rv, mv = bufs[half]
                rn, mn = bufs[1 - half]

                def prefetch_idx():
                    pltpu.async_copy(
                        row4_hbm.at[q * NW + g0 + w + 1], rn, isem
                    )

                def launch_next():
                    pltpu.make_async_copy(row4_hbm.at[0], rn, isem).wait()
                    fire_gathers(rn, mn)

                if half == 0:
                    prefetch_idx()
                    drain(gsem, mv)
                    @pl.when(i > 0)
                    def _():
                        drain(ssem, mn)
                    launch_next()
                else:
                    has_next = i < (WPT // 2 - 1)
                    @pl.when(has_next)
                    def _():
                        prefetch_idx()
                    drain(gsem, mv)
                    drain(ssem, mn)
                    @pl.when(has_next)
                    def _():
                        launch_next()
                fire_scatters(w, mv)
            return carry

        lax.fori_loop(0, WPT // 2, body2, 0)
        drain(ssem, msg_v1)

    def zero_acc():
        pltpu.sync_copy(ztile_hbm, msg_v0)
        for k in range(NFULL):
            pltpu.sync_copy(msg_v0, acc_sh.at[pl.ds(base + k * W, W), :])
        pltpu.sync_copy(
            msg_v0.at[pl.ds(0, REM), :],
            acc_sh.at[pl.ds(base + NFULL * W, REM), :],
        )

    def writeback_scaled(acc_out, q):
        obase = q * N_PAD + base

        def wb_chunk(k, carry, rows):
            lo = base + k * W
            olo = obase + k * W
            pltpu.sync_copy(
                acc_sh.at[pl.ds(lo, rows), :], msg_v0.at[pl.ds(0, rows), :]
            )
            pltpu.sync_copy(
                msg_v0.at[pl.ds(0, rows), :],
                acc_out.at[pl.ds(olo, rows), :],
            )

            def mul2(r):
                b = bcast(k * W + r)
                return msg_v0[r, :] * (b * b)

            scale_rows(rows, mul2)
            pltpu.sync_copy(
                msg_v1.at[pl.ds(0, rows), :],
                xs_scr.at[pl.ds(olo, rows), :],
            )
            return carry

        lax.fori_loop(0, NFULL, functools.partial(wb_chunk, rows=W), 0)
        wb_chunk(NFULL, 0, rows=REM)

    def writeback_final(q):
        obase = q * N_PAD + base

        def wb_chunk(k, carry, rows):
            lo = base + k * W
            olo = obase + k * W
            pltpu.sync_copy(
                acc_sh.at[pl.ds(lo, rows), :], msg_v0.at[pl.ds(0, rows), :]
            )
            pltpu.sync_copy(
                acc1_out.at[pl.ds(olo, rows), :],
                msg_v1.at[pl.ds(0, rows), :],
            )
            scale_rows(rows, lambda r: msg_v0[r, :] + msg_v1[r, :])
            pltpu.sync_copy(
                acc2_out.at[pl.ds(olo, rows), :],
                msg_v0.at[pl.ds(0, rows), :],
            )

            def add2(r):
                return (msg_v0[r, :] + msg_v1[r, :]) * (
                    bcast(k * W + r) * third
                )

            scale_rows(rows, add2)
            pltpu.sync_copy(
                msg_v1.at[pl.ds(0, rows), :],
                out_hbm.at[pl.ds(lo, rows), pl.ds(q * KQ, KQ)],
            )
            return carry

        lax.fori_loop(0, NFULL, functools.partial(wb_chunk, rows=W), 0)
        wb_chunk(NFULL, 0, rows=REM)

    def layer_pass(t, carry):
        ell = t // 2
        q = 2 * c + (t % 2)
        zero_acc()
        plsc.subcore_barrier()
        scatter_pass(q)
        plsc.subcore_barrier()

        @pl.when(ell == 0)
        def _():
            writeback_scaled(acc1_out, q)

        @pl.when(ell == 1)
        def _():
            writeback_scaled(acc2_out, q)

        @pl.when(ell == 2)
        def _():
            writeback_final(q)
        return carry

    lax.fori_loop(0, 6, layer_pass, 0)


def kernel(Gu, Gi, edge_index):
    x0 = jnp.concatenate([Gu, Gi], axis=0)
    x0p = jnp.pad(x0, ((0, N_PAD - N_NODES), (0, 0)))

    row = edge_index[0]
    col = edge_index[1]
    npad = E_PAD - E
    i = jnp.arange(npad, dtype=jnp.int32)
    prow = i % N_NODES
    pcol = N_NODES + (i % DUMP)
    rowp = jnp.concatenate([row, prow])
    colp = jnp.concatenate([col, pcol]).reshape(NW, NS, 128)
    row4 = jnp.stack([rowp + q * N_PAD for q in range(NQ)])
    row4 = row4.reshape(NQ * NW, NS, 128)

    zrow = jnp.zeros((TILE_ROWS,), jnp.float32)
    ztile = jnp.zeros((W, KQ), jnp.float32)

    out, _, _, _ = _main_kernel(x0p, row4, colp, ztile, zrow)
    return out[:N_USERS], out[N_USERS:N_NODES]


# speedup vs baseline: 19.2265x; 1.0013x over previous
import functools

import jax
import jax.numpy as jnp
from jax import lax
from jax.experimental import pallas as pl
from jax.experimental.pallas import tpu as pltpu
from jax.experimental.pallas import tpu_sc as plsc

N_USERS = 10000
N_NODES = 50000
N_PAD = 50176
DUMP = N_PAD - N_NODES
K = 64
KQ = 16
NQ = 4
E = 800000
W = 512
NS = W // 128
NW = 1600
E_PAD = NW * W
WPT = NW // 16
TILE_ROWS = N_PAD // 16
NFULL = TILE_ROWS // W
REM = TILE_ROWS - NFULL * W

_mesh = plsc.VectorSubcoreMesh(core_axis_name="c", subcore_axis_name="s")
_sc_params = pltpu.CompilerParams(use_tc_tiling_on_sc=False,
                                  needs_layout_passes=False)


@functools.partial(
    pl.kernel,
    out_type=[
        jax.ShapeDtypeStruct((N_PAD, K), jnp.float32),
        jax.ShapeDtypeStruct((NQ * N_PAD, KQ), jnp.float32),
        jax.ShapeDtypeStruct((NQ * N_PAD, KQ), jnp.float32),
        jax.ShapeDtypeStruct((NQ * N_PAD, KQ), jnp.float32),
    ],
    mesh=_mesh,
    compiler_params=_sc_params,
    scratch_types=[
        pltpu.VMEM((NS, 128), jnp.int32),
        pltpu.VMEM((NS, 128), jnp.int32),
        pltpu.VMEM((WPT, NS, 128), jnp.int32),
        pltpu.VMEM((W, KQ), jnp.float32),
        pltpu.VMEM((W, KQ), jnp.float32),
        pltpu.VMEM((NS, 128), jnp.float32),
        pltpu.VMEM((TILE_ROWS,), jnp.float32),
        pltpu.VMEM((TILE_ROWS,), jnp.float32),
        pltpu.VMEM_SHARED((N_PAD, KQ), jnp.float32),
        pltpu.VMEM_SHARED((N_PAD,), jnp.float32),
        pltpu.SemaphoreType.DMA,
        pltpu.SemaphoreType.DMA,
        pltpu.SemaphoreType.DMA,
    ],
)
def _main_kernel(x0_hbm, row4_hbm, col_hbm, ztile_hbm, zrow_hbm,
                 out_hbm, xs_scr, acc1_out, acc2_out,
                 row_v0, row_v1, col_all, msg_v0, msg_v1, ones_v,
                 stage_v, dinv_v, acc_sh, deg_sh,
                 gsem, ssem, isem):
    c = lax.axis_index("c")
    s = lax.axis_index("s")
    base = s * TILE_ROWS
    g0 = s * WPT
    third = jnp.float32(1.0 / 3.0)

    pltpu.sync_copy(col_hbm.at[pl.ds(g0, WPT)], col_all)

    for j in range(NS):
        for k in range(8):
            ones_v[j, pl.ds(k * 16, 16)] = jnp.full((16,), 1.0, jnp.float32)
    pltpu.sync_copy(zrow_hbm, stage_v)
    pltpu.sync_copy(stage_v, deg_sh.at[pl.ds(base, TILE_ROWS)])
    plsc.subcore_barrier()

    def deg_drain():
        for _ in range(NS):
            pltpu.make_async_copy(zrow_hbm.at[pl.ds(0, 128)],
                                  ones_v.at[0], ssem).wait()

    def deg_body(w, carry):
        @pl.when(w > 0)
        def _():
            deg_drain()
        for j in range(NS):
            pltpu.async_copy(ones_v.at[j], deg_sh.at[col_all.at[w, j]],
                             ssem, add=True)
        return carry

    lax.fori_loop(0, WPT, deg_body, 0)
    deg_drain()
    plsc.subcore_barrier()

    pltpu.sync_copy(deg_sh.at[pl.ds(base, TILE_ROWS)], stage_v)

    def rsqrt_body(i, carry):
        x = stage_v[pl.ds(i * 16, 16)]
        h = x * jnp.float32(0.5)
        yi = jnp.int32(0x5F3759DF) - (plsc.bitcast(x, jnp.int32) >> 1)
        y = plsc.bitcast(yi, jnp.float32)
        for _ in range(3):
            y = y * (jnp.float32(1.5) - h * y * y)
        dinv_v[pl.ds(i * 16, 16)] = jnp.where(x > 0, y, jnp.float32(0.0))
        return carry

    lax.fori_loop(0, TILE_ROWS // 16, rsqrt_body, 0)

    def bcast(idx):
        return plsc.load_gather(dinv_v, [jnp.full((16,), 0, jnp.int32) + idx])

    def scale_rows(rows, mul_fn):
        def grp(g, carry):
            for u in range(16):
                r = g * 16 + u
                msg_v1[r, :] = mul_fn(r)
            return carry
        lax.fori_loop(0, rows // 16, grp, 0)

    def xs1_pass(p, carry):
        q = 2 * c + p

        def xs1_chunk(k, carry2, rows):
            lo = base + k * W
            pltpu.sync_copy(
                x0_hbm.at[pl.ds(lo, rows), pl.ds(q * KQ, KQ)],
                msg_v0.at[pl.ds(0, rows), :],
            )
            scale_rows(rows, lambda r: msg_v0[r, :] * bcast(k * W + r))
            pltpu.sync_copy(
                msg_v1.at[pl.ds(0, rows), :],
                xs_scr.at[pl.ds(q * N_PAD + lo, rows), :],
            )
            return carry2

        lax.fori_loop(0, NFULL, functools.partial(xs1_chunk, rows=W), 0)
        xs1_chunk(NFULL, 0, rows=REM)
        return carry

    lax.fori_loop(0, 2, xs1_pass, 0)

    def fire_gathers(rv, mv):
        for j in range(NS):
            pltpu.async_copy(
                xs_scr.at[rv.at[j]], mv.at[pl.ds(j * 128, 128), :], gsem
            )

    def fire_scatters(w, mv):
        for j in range(NS):
            pltpu.async_copy(
                mv.at[pl.ds(j * 128, 128), :],
                acc_sh.at[col_all.at[w, j]], ssem, add=True,
            )

    def drain(sem, dst):
        pltpu.make_async_copy(x0_hbm.at[pl.ds(0, W), pl.ds(0, KQ)], dst,
                              sem).wait()

    def scatter_pass(q):
        pltpu.sync_copy(row4_hbm.at[q * NW + g0], row_v0)
        fire_gathers(row_v0, msg_v0)

        def body2(i, carry):
            bufs = ((row_v0, msg_v0), (row_v1, msg_v1))
            for half in (0, 1):
                w = 2 * i + half
                rv, mv = bufs[half]
                rn, mn = bufs[1 - half]

                def prefetch_idx():
                    pltpu.async_copy(
                        row4_hbm.at[q * NW + g0 + w + 1], rn, isem
                    )

                def launch_next():
                    pltpu.make_async_copy(row4_hbm.at[0], rn, isem).wait()
                    fire_gathers(rn, mn)

                if half == 0:
                    prefetch_idx()
                    drain(gsem, mv)
                    @pl.when(i > 0)
                    def _():
                        drain(ssem, mn)
                    launch_next()
                else:
                    has_next = i < (WPT // 2 - 1)
                    @pl.when(has_next)
                    def _():
                        prefetch_idx()
                    drain(gsem, mv)
                    drain(ssem, mn)
                    @pl.when(has_next)
                    def _():
                        launch_next()
                fire_scatters(w, mv)
            return carry

        lax.fori_loop(0, WPT // 2, body2, 0)
        drain(ssem, msg_v1)

    def zero_acc():
        pltpu.sync_copy(ztile_hbm, msg_v0)
        hs = [
            pltpu.async_copy(msg_v0, acc_sh.at[pl.ds(base + k * W, W), :],
                             gsem)
            for k in range(NFULL)
        ]
        hs.append(
            pltpu.async_copy(
                msg_v0.at[pl.ds(0, REM), :],
                acc_sh.at[pl.ds(base + NFULL * W, REM), :], gsem,
            )
        )
        for h in hs:
            h.wait()

    def writeback_scaled(acc_out, q):
        obase = q * N_PAD + base

        def wb_chunk(k, carry, rows):
            lo = base + k * W
            olo = obase + k * W
            pltpu.sync_copy(
                acc_sh.at[pl.ds(lo, rows), :], msg_v0.at[pl.ds(0, rows), :]
            )
            pltpu.sync_copy(
                msg_v0.at[pl.ds(0, rows), :],
                acc_out.at[pl.ds(olo, rows), :],
            )

            def mul2(r):
                b = bcast(k * W + r)
                return msg_v0[r, :] * (b * b)

            scale_rows(rows, mul2)
            pltpu.sync_copy(
                msg_v1.at[pl.ds(0, rows), :],
                xs_scr.at[pl.ds(olo, rows), :],
            )
            return carry

        lax.fori_loop(0, NFULL, functools.partial(wb_chunk, rows=W), 0)
        wb_chunk(NFULL, 0, rows=REM)

    def writeback_final(q):
        obase = q * N_PAD + base

        def wb_chunk(k, carry, rows):
            lo = base + k * W
            olo = obase + k * W
            pltpu.sync_copy(
                acc_sh.at[pl.ds(lo, rows), :], msg_v0.at[pl.ds(0, rows), :]
            )
            pltpu.sync_copy(
                acc1_out.at[pl.ds(olo, rows), :],
                msg_v1.at[pl.ds(0, rows), :],
            )
            scale_rows(rows, lambda r: msg_v0[r, :] + msg_v1[r, :])
            pltpu.sync_copy(
                acc2_out.at[pl.ds(olo, rows), :],
                msg_v0.at[pl.ds(0, rows), :],
            )

            def add2(r):
                return (msg_v0[r, :] + msg_v1[r, :]) * (
                    bcast(k * W + r) * third
                )

            scale_rows(rows, add2)
            pltpu.sync_copy(
                msg_v1.at[pl.ds(0, rows), :],
                out_hbm.at[pl.ds(lo, rows), pl.ds(q * KQ, KQ)],
            )
            return carry

        lax.fori_loop(0, NFULL, functools.partial(wb_chunk, rows=W), 0)
        wb_chunk(NFULL, 0, rows=REM)

    def layer_pass(t, carry):
        ell = t // 2
        q = 2 * c + (t % 2)
        zero_acc()
        plsc.subcore_barrier()
        scatter_pass(q)
        plsc.subcore_barrier()

        @pl.when(ell == 0)
        def _():
            writeback_scaled(acc1_out, q)

        @pl.when(ell == 1)
        def _():
            writeback_scaled(acc2_out, q)

        @pl.when(ell == 2)
        def _():
            writeback_final(q)
        return carry

    lax.fori_loop(0, 6, layer_pass, 0)


def kernel(Gu, Gi, edge_index):
    x0 = jnp.concatenate([Gu, Gi], axis=0)
    x0p = jnp.pad(x0, ((0, N_PAD - N_NODES), (0, 0)))

    row = edge_index[0]
    col = edge_index[1]
    npad = E_PAD - E
    i = jnp.arange(npad, dtype=jnp.int32)
    prow = i % N_NODES
    pcol = N_NODES + (i % DUMP)
    rowp = jnp.concatenate([row, prow])
    colp = jnp.concatenate([col, pcol]).reshape(NW, NS, 128)
    row4 = jnp.stack([rowp + q * N_PAD for q in range(NQ)])
    row4 = row4.reshape(NQ * NW, NS, 128)

    zrow = jnp.zeros((TILE_ROWS,), jnp.float32)
    ztile = jnp.zeros((W, KQ), jnp.float32)

    out, _, _, _ = _main_kernel(x0p, row4, colp, ztile, zrow)
    return out[:N_USERS], out[N_USERS:N_NODES]


# speedup vs baseline: 19.2451x; 1.0010x over previous
import functools

import jax
import jax.numpy as jnp
from jax import lax
from jax.experimental import pallas as pl
from jax.experimental.pallas import tpu as pltpu
from jax.experimental.pallas import tpu_sc as plsc

N_USERS = 10000
N_NODES = 50000
N_PAD = 50176
DUMP = N_PAD - N_NODES
K = 64
KQ = 16
NQ = 4
E = 800000
W = 512
NS = W // 128
NW = 1600
E_PAD = NW * W
WPT = NW // 16
TILE_ROWS = N_PAD // 16
NFULL = TILE_ROWS // W
REM = TILE_ROWS - NFULL * W

_mesh = plsc.VectorSubcoreMesh(core_axis_name="c", subcore_axis_name="s")
_sc_params = pltpu.CompilerParams(use_tc_tiling_on_sc=False,
                                  needs_layout_passes=False)


@functools.partial(
    pl.kernel,
    out_type=[
        jax.ShapeDtypeStruct((N_PAD, K), jnp.float32),
        jax.ShapeDtypeStruct((NQ * N_PAD, KQ), jnp.float32),
        jax.ShapeDtypeStruct((NQ * N_PAD, KQ), jnp.float32),
        jax.ShapeDtypeStruct((NQ * N_PAD, KQ), jnp.float32),
    ],
    mesh=_mesh,
    compiler_params=_sc_params,
    scratch_types=[
        pltpu.VMEM((NS, 128), jnp.int32),
        pltpu.VMEM((NS, 128), jnp.int32),
        pltpu.VMEM((NS, 128), jnp.int32),
        pltpu.VMEM((NS, 128), jnp.int32),
        pltpu.VMEM((WPT, NS, 128), jnp.int32),
        pltpu.VMEM((W, KQ), jnp.float32),
        pltpu.VMEM((W, KQ), jnp.float32),
        pltpu.VMEM((NS, 128), jnp.float32),
        pltpu.VMEM((TILE_ROWS,), jnp.float32),
        pltpu.VMEM((TILE_ROWS,), jnp.float32),
        pltpu.VMEM_SHARED((N_PAD, KQ), jnp.float32),
        pltpu.VMEM_SHARED((N_PAD,), jnp.float32),
        pltpu.SemaphoreType.DMA,
        pltpu.SemaphoreType.DMA,
        pltpu.SemaphoreType.DMA,
    ],
)
def _main_kernel(x0_hbm, row4_hbm, col_hbm, ztile_hbm, zrow_hbm,
                 out_hbm, xs_scr, acc1_out, acc2_out,
                 row_v0, row_v1, row_v2, row_v3, col_all, msg_v0, msg_v1,
                 ones_v,
                 stage_v, dinv_v, acc_sh, deg_sh,
                 gsem, ssem, isem):
    c = lax.axis_index("c")
    s = lax.axis_index("s")
    base = s * TILE_ROWS
    g0 = s * WPT
    third = jnp.float32(1.0 / 3.0)

    pltpu.sync_copy(col_hbm.at[pl.ds(g0, WPT)], col_all)

    for j in range(NS):
        for k in range(8):
            ones_v[j, pl.ds(k * 16, 16)] = jnp.full((16,), 1.0, jnp.float32)
    pltpu.sync_copy(zrow_hbm, stage_v)
    pltpu.sync_copy(stage_v, deg_sh.at[pl.ds(base, TILE_ROWS)])
    plsc.subcore_barrier()

    def deg_drain():
        for _ in range(NS):
            pltpu.make_async_copy(zrow_hbm.at[pl.ds(0, 128)],
                                  ones_v.at[0], ssem).wait()

    def deg_body(w, carry):
        @pl.when(w > 0)
        def _():
            deg_drain()
        for j in range(NS):
            pltpu.async_copy(ones_v.at[j], deg_sh.at[col_all.at[w, j]],
                             ssem, add=True)
        return carry

    lax.fori_loop(0, WPT, deg_body, 0)
    deg_drain()
    plsc.subcore_barrier()

    pltpu.sync_copy(deg_sh.at[pl.ds(base, TILE_ROWS)], stage_v)

    def rsqrt_body(i, carry):
        x = stage_v[pl.ds(i * 16, 16)]
        h = x * jnp.float32(0.5)
        yi = jnp.int32(0x5F3759DF) - (plsc.bitcast(x, jnp.int32) >> 1)
        y = plsc.bitcast(yi, jnp.float32)
        for _ in range(3):
            y = y * (jnp.float32(1.5) - h * y * y)
        dinv_v[pl.ds(i * 16, 16)] = jnp.where(x > 0, y, jnp.float32(0.0))
        return carry

    lax.fori_loop(0, TILE_ROWS // 16, rsqrt_body, 0)

    def bcast(idx):
        return plsc.load_gather(dinv_v, [jnp.full((16,), 0, jnp.int32) + idx])

    def scale_rows(rows, mul_fn):
        def grp(g, carry):
            for u in range(16):
                r = g * 16 + u
                msg_v1[r, :] = mul_fn(r)
            return carry
        lax.fori_loop(0, rows // 16, grp, 0)

    def xs1_pass(p, carry):
        q = 2 * c + p

        def xs1_chunk(k, carry2, rows):
            lo = base + k * W
            pltpu.sync_copy(
                x0_hbm.at[pl.ds(lo, rows), pl.ds(q * KQ, KQ)],
                msg_v0.at[pl.ds(0, rows), :],
            )
            scale_rows(rows, lambda r: msg_v0[r, :] * bcast(k * W + r))
            pltpu.sync_copy(
                msg_v1.at[pl.ds(0, rows), :],
                xs_scr.at[pl.ds(q * N_PAD + lo, rows), :],
            )
            return carry2

        lax.fori_loop(0, NFULL, functools.partial(xs1_chunk, rows=W), 0)
        xs1_chunk(NFULL, 0, rows=REM)
        return carry

    lax.fori_loop(0, 2, xs1_pass, 0)

    def fire_gathers(rv, mv):
        for j in range(NS):
            pltpu.async_copy(
                xs_scr.at[rv.at[j]], mv.at[pl.ds(j * 128, 128), :], gsem
            )

    def fire_scatters(w, mv):
        for j in range(NS):
            pltpu.async_copy(
                mv.at[pl.ds(j * 128, 128), :],
                acc_sh.at[col_all.at[w, j]], ssem, add=True,
            )

    def drain(sem, dst):
        pltpu.make_async_copy(x0_hbm.at[pl.ds(0, W), pl.ds(0, KQ)], dst,
                              sem).wait()

    def scatter_pass(q):
        rows_ = (row_v0, row_v1, row_v2, row_v3)
        msgs = (msg_v0, msg_v1)
        pltpu.sync_copy(row4_hbm.at[q * NW + g0], row_v0)
        pltpu.async_copy(row4_hbm.at[q * NW + g0 + 1], row_v1, isem)
        fire_gathers(row_v0, msg_v0)

        def body4(i, carry):
            for half in (0, 1, 2, 3):
                w = 4 * i + half
                rv, mv = rows_[half], msgs[half % 2]
                rn, mn = rows_[(half + 1) % 4], msgs[(half + 1) % 2]
                r2 = rows_[(half + 2) % 4]

                def prefetch_idx2():
                    pltpu.async_copy(
                        row4_hbm.at[q * NW + g0 + w + 2], r2, isem
                    )

                def launch_next():
                    pltpu.make_async_copy(row4_hbm.at[0], rn, isem).wait()
                    fire_gathers(rn, mn)

                have2 = half < 2 or i < (WPT // 4 - 1)
                have1 = half < 3 or i < (WPT // 4 - 1)
                if have2 is True:
                    prefetch_idx2()
                else:
                    @pl.when(have2)
                    def _():
                        prefetch_idx2()
                drain(gsem, mv)
                if half == 0:
                    @pl.when(i > 0)
                    def _():
                        drain(ssem, mn)
                else:
                    drain(ssem, mn)
                if have1 is True:
                    launch_next()
                else:
                    @pl.when(have1)
                    def _():
                        launch_next()
                fire_scatters(w, mv)
            return carry

        lax.fori_loop(0, WPT // 4, body4, 0)
        drain(ssem, msg_v1)

    def zero_acc():
        pltpu.sync_copy(ztile_hbm, msg_v0)
        hs = [
            pltpu.async_copy(msg_v0, acc_sh.at[pl.ds(base + k * W, W), :],
                             gsem)
            for k in range(NFULL)
        ]
        hs.append(
            pltpu.async_copy(
                msg_v0.at[pl.ds(0, REM), :],
                acc_sh.at[pl.ds(base + NFULL * W, REM), :], gsem,
            )
        )
        for h in hs:
            h.wait()

    def writeback_scaled(acc_out, q):
        obase = q * N_PAD + base

        def wb_chunk(k, carry, rows):
            lo = base + k * W
            olo = obase + k * W
            pltpu.sync_copy(
                acc_sh.at[pl.ds(lo, rows), :], msg_v0.at[pl.ds(0, rows), :]
            )
            pltpu.sync_copy(
                msg_v0.at[pl.ds(0, rows), :],
                acc_out.at[pl.ds(olo, rows), :],
            )

            def mul2(r):
                b = bcast(k * W + r)
                return msg_v0[r, :] * (b * b)

            scale_rows(rows, mul2)
            pltpu.sync_copy(
                msg_v1.at[pl.ds(0, rows), :],
                xs_scr.at[pl.ds(olo, rows), :],
            )
            return carry

        lax.fori_loop(0, NFULL, functools.partial(wb_chunk, rows=W), 0)
        wb_chunk(NFULL, 0, rows=REM)

    def writeback_final(q):
        obase = q * N_PAD + base

        def wb_chunk(k, carry, rows):
            lo = base + k * W
            olo = obase + k * W
            pltpu.sync_copy(
                acc_sh.at[pl.ds(lo, rows), :], msg_v0.at[pl.ds(0, rows), :]
            )
            pltpu.sync_copy(
                acc1_out.at[pl.ds(olo, rows), :],
                msg_v1.at[pl.ds(0, rows), :],
            )
            scale_rows(rows, lambda r: msg_v0[r, :] + msg_v1[r, :])
            pltpu.sync_copy(
                acc2_out.at[pl.ds(olo, rows), :],
                msg_v0.at[pl.ds(0, rows), :],
            )

            def add2(r):
                return (msg_v0[r, :] + msg_v1[r, :]) * (
                    bcast(k * W + r) * third
                )

            scale_rows(rows, add2)
            pltpu.sync_copy(
                msg_v1.at[pl.ds(0, rows), :],
                out_hbm.at[pl.ds(lo, rows), pl.ds(q * KQ, KQ)],
            )
            return carry

        lax.fori_loop(0, NFULL, functools.partial(wb_chunk, rows=W), 0)
        wb_chunk(NFULL, 0, rows=REM)

    def layer_pass(t, carry):
        ell = t // 2
        q = 2 * c + (t % 2)
        zero_acc()
        plsc.subcore_barrier()
        scatter_pass(q)
        plsc.subcore_barrier()

        @pl.when(ell == 0)
        def _():
            writeback_scaled(acc1_out, q)

        @pl.when(ell == 1)
        def _():
            writeback_scaled(acc2_out, q)

        @pl.when(ell == 2)
        def _():
            writeback_final(q)
        return carry

    lax.fori_loop(0, 6, layer_pass, 0)


def kernel(Gu, Gi, edge_index):
    x0 = jnp.concatenate([Gu, Gi], axis=0)
    x0p = jnp.pad(x0, ((0, N_PAD - N_NODES), (0, 0)))

    row = edge_index[0]
    col = edge_index[1]
    npad = E_PAD - E
    i = jnp.arange(npad, dtype=jnp.int32)
    prow = i % N_NODES
    pcol = N_NODES + (i % DUMP)
    rowp = jnp.concatenate([row, prow])
    colp = jnp.concatenate([col, pcol]).reshape(NW, NS, 128)
    row4 = jnp.stack([rowp + q * N_PAD for q in range(NQ)])
    row4 = row4.reshape(NQ * NW, NS, 128)

    zrow = jnp.zeros((TILE_ROWS,), jnp.float32)
    ztile = jnp.zeros((W, KQ), jnp.float32)

    out, _, _, _ = _main_kernel(x0p, row4, colp, ztile, zrow)
    return out[:N_USERS], out[N_USERS:N_NODES]
